# baseline (speedup 1.0000x reference)
import functools

import jax
import jax.numpy as jnp
from jax import lax
from jax.experimental import pallas as pl
from jax.experimental.pallas import tpu as pltpu
from jax.experimental.pallas import tpu_sc as plsc

B = 4096
L = 200
DIM = 64
PAD = 128
N = B * L
NK = DIM // 16
NB = 4

NC = 2
NS = 16
NW = NC * NS
ROWS_W = B // NW

_mesh = plsc.VectorSubcoreMesh(core_axis_name="c", subcore_axis_name="s")


@functools.partial(
    pl.kernel,
    mesh=_mesh,
    scratch_types=[
        [pltpu.VMEM((L,), jnp.int32)] * NB,
        [pltpu.VMEM((L,), jnp.int32)] * NB,
        [pltpu.VMEM((L, PAD), jnp.float32)] * NB,
        pltpu.VMEM_SHARED((L, PAD), jnp.float32),
        pltpu.VMEM((DIM,), jnp.float32),
        pltpu.VMEM((DIM,), jnp.float32),
        [pltpu.SemaphoreType.DMA] * NB,
        [pltpu.SemaphoreType.DMA] * NB,
        [pltpu.SemaphoreType.DMA] * NB,
        [pltpu.SemaphoreType.DMA] * NB,
    ],
)
def _embed_ln_kernel(tok_hbm, pos_hbm, ttab_hbm, ptab_hbm, gamma_hbm, beta_hbm,
                     out_hbm,
                     idxt, idxp, erows, ptab_sh,
                     gamma_v, beta_v, sem_i, sem_p, sem_t, sem_o):
    sid = lax.axis_index("s")
    wid = sid * NC + lax.axis_index("c")
    base_b = wid * ROWS_W

    pltpu.sync_copy(gamma_hbm, gamma_v)
    pltpu.sync_copy(beta_hbm, beta_v)

    @pl.when(sid == 0)
    def _():
        pltpu.sync_copy(ptab_hbm, erows[0])
        pltpu.sync_copy(erows[0], ptab_sh)

    plsc.subcore_barrier()

    lane = lax.iota(jnp.int32, 16)
    perms = [lane ^ sh for sh in (1, 2, 4, 8)]

    g = [gamma_v[pl.ds(k * 16, 16)] for k in range(NK)]
    bt = [beta_v[pl.ds(k * 16, 16)] for k in range(NK)]

    def allsum(v):
        for p in perms:
            v = v + v.at[p].get(mode="promise_in_bounds")
        return v

    def stage_idx(gi, b):
        bi = base_b + gi
        pltpu.async_copy(tok_hbm.at[pl.ds(bi * L, L)], idxt[b], sem_i[b])
        pltpu.async_copy(pos_hbm.at[pl.ds(bi * L, L)], idxp[b], sem_i[b])

    def wait_idx(b):
        pltpu.make_async_copy(tok_hbm.at[pl.ds(0, L)], idxt[b],
                              sem_i[b]).wait()
        pltpu.make_async_copy(pos_hbm.at[pl.ds(0, L)], idxp[b],
                              sem_i[b]).wait()

    def stage_pos(gi, b):
        @pl.when(gi >= NB)
        def _():
            pltpu.make_async_copy(erows[b], out_hbm.at[base_b + gi],
                                  sem_o[b]).wait()
        wait_idx(b)
        pltpu.async_copy(ptab_sh.at[idxp[b]], erows[b], sem_p[b])

    def stage_tok(b):
        pltpu.make_async_copy(ptab_sh.at[idxp[b]], erows[b], sem_p[b]).wait()
        pltpu.async_copy(ttab_hbm.at[idxt[b]], erows[b], sem_t[b], add=True)

    def compute_chunk(b):
        @plsc.parallel_loop(0, L, unroll=4)
        def tok_body(t):
            e = [erows[b][t, pl.ds(k * 16, 16)] for k in range(NK)]
            s = (e[0] + e[1]) + (e[2] + e[3])
            q = (e[0] * e[0] + e[1] * e[1]) + (e[2] * e[2] + e[3] * e[3])
            mean = allsum(s) * (1.0 / DIM)
            var = allsum(q) * (1.0 / DIM) - mean * mean
            xv = jnp.maximum(var, 0.0) + 1e-12
            iv = lax.bitcast_convert_type(xv, jnp.int32)
            iv = 0x5F3759DF - (iv >> 1)
            y = lax.bitcast_convert_type(iv, jnp.float32)
            hx = xv * 0.5
            for _ in range(2):
                y = y * (1.5 - hx * y * y)
            for k in range(NK):
                erows[b][t, pl.ds(k * 16, 16)] = \
                    (e[k] - mean) * y * g[k] + bt[k]

    stage_idx(0, 0)
    stage_idx(1, 1)
    stage_idx(2, 2)
    stage_pos(0, 0)
    stage_pos(1, 1)
    stage_tok(0)

    def body4(ch, carry):
        for j in range(NB):
            gi = NB * ch + j

            @pl.when(gi + 3 < ROWS_W)
            def _():
                stage_idx(gi + 3, (j + 3) % NB)

            @pl.when(gi + 2 < ROWS_W)
            def _():
                stage_pos(gi + 2, (j + 2) % NB)

            @pl.when(gi + 1 < ROWS_W)
            def _():
                stage_tok((j + 1) % NB)

            pltpu.make_async_copy(ttab_hbm.at[idxt[j]], erows[j],
                                  sem_t[j]).wait()
            compute_chunk(j)
            pltpu.async_copy(erows[j], out_hbm.at[base_b + gi], sem_o[j])
        return carry

    lax.fori_loop(0, ROWS_W // NB, body4, 0)

    for b in range(NB):
        pltpu.make_async_copy(erows[b], out_hbm.at[base_b], sem_o[b]).wait()


def kernel(input_token_id, input_position_id, token_table, pos_table,
           ln_gamma, ln_beta):
    tok = jnp.asarray(input_token_id, jnp.int32).reshape(N)
    pos = jnp.asarray(input_position_id, jnp.int32).reshape(N)
    ttab = jnp.pad(token_table, ((0, 0), (0, PAD - DIM)))
    ptab = jnp.pad(pos_table[:L], ((0, 0), (0, PAD - DIM)))
    buf = jax.new_ref(jnp.zeros((B, L, PAD), jnp.float32))
    _embed_ln_kernel(tok, pos, ttab, ptab, ln_gamma, ln_beta, buf)
    return buf[...][:, :, :DIM]


# speedup vs baseline: 1.2229x; 1.2229x over previous
import functools

import jax
import jax.numpy as jnp
from jax import lax
from jax.experimental import pallas as pl
from jax.experimental.pallas import tpu as pltpu
from jax.experimental.pallas import tpu_sc as plsc

B = 4096
L = 200
DIM = 64
PAD = 128
N = B * L
NK = DIM // 16
NB = 4

NC = 2
NS = 16
NW = NC * NS
ROWS_W = B // NW

_mesh = plsc.VectorSubcoreMesh(core_axis_name="c", subcore_axis_name="s")


@functools.partial(
    pl.kernel,
    out_type=jax.ShapeDtypeStruct((B, L, PAD), jnp.float32),
    mesh=_mesh,
    scratch_types=[
        [pltpu.VMEM((L,), jnp.int32)] * NB,
        [pltpu.VMEM((L,), jnp.int32)] * NB,
        [pltpu.VMEM((L, PAD), jnp.float32)] * NB,
        pltpu.VMEM_SHARED((L, PAD), jnp.float32),
        pltpu.VMEM((DIM,), jnp.float32),
        pltpu.VMEM((DIM,), jnp.float32),
        [pltpu.SemaphoreType.DMA] * NB,
        [pltpu.SemaphoreType.DMA] * NB,
        [pltpu.SemaphoreType.DMA] * NB,
        [pltpu.SemaphoreType.DMA] * NB,
    ],
)
def _embed_ln_kernel(tok_hbm, pos_hbm, ttab_hbm, ptab_hbm, gamma_hbm, beta_hbm,
                     out_hbm,
                     idxt, idxp, erows, ptab_sh,
                     gamma_v, beta_v, sem_i, sem_p, sem_t, sem_o):
    sid = lax.axis_index("s")
    wid = sid * NC + lax.axis_index("c")
    base_b = wid * ROWS_W

    pltpu.sync_copy(gamma_hbm, gamma_v)
    pltpu.sync_copy(beta_hbm, beta_v)

    @pl.when(sid == 0)
    def _():
        pltpu.sync_copy(ptab_hbm, erows[0])
        pltpu.sync_copy(erows[0], ptab_sh)

    plsc.subcore_barrier()

    lane = lax.iota(jnp.int32, 16)
    perms = [lane ^ sh for sh in (1, 2, 4, 8)]

    g = [gamma_v[pl.ds(k * 16, 16)] for k in range(NK)]
    bt = [beta_v[pl.ds(k * 16, 16)] for k in range(NK)]

    def allsum(v):
        for p in perms:
            v = v + v.at[p].get(mode="promise_in_bounds")
        return v

    def stage_idx(gi, b):
        bi = base_b + gi
        pltpu.async_copy(tok_hbm.at[bi], idxt[b], sem_i[b])
        pltpu.async_copy(pos_hbm.at[bi], idxp[b], sem_i[b])

    def wait_idx(b):
        pltpu.make_async_copy(tok_hbm.at[0], idxt[b], sem_i[b]).wait()
        pltpu.make_async_copy(pos_hbm.at[0], idxp[b], sem_i[b]).wait()

    def stage_pos(gi, b):
        @pl.when(gi >= NB)
        def _():
            pltpu.make_async_copy(erows[b], out_hbm.at[base_b + gi],
                                  sem_o[b]).wait()
        wait_idx(b)
        pltpu.async_copy(ptab_sh.at[idxp[b]], erows[b], sem_p[b])

    def stage_tok(b):
        pltpu.make_async_copy(ptab_sh.at[idxp[b]], erows[b], sem_p[b]).wait()
        pltpu.async_copy(ttab_hbm.at[idxt[b]], erows[b], sem_t[b], add=True)

    def compute_chunk(b):
        @plsc.parallel_loop(0, L, unroll=4)
        def tok_body(t):
            e = [erows[b][t, pl.ds(k * 16, 16)] for k in range(NK)]
            s = (e[0] + e[1]) + (e[2] + e[3])
            q = (e[0] * e[0] + e[1] * e[1]) + (e[2] * e[2] + e[3] * e[3])
            mean = allsum(s) * (1.0 / DIM)
            var = allsum(q) * (1.0 / DIM) - mean * mean
            xv = jnp.maximum(var, 0.0) + 1e-12
            iv = lax.bitcast_convert_type(xv, jnp.int32)
            iv = 0x5F3759DF - (iv >> 1)
            y = lax.bitcast_convert_type(iv, jnp.float32)
            hx = xv * 0.5
            for _ in range(2):
                y = y * (1.5 - hx * y * y)
            for k in range(NK):
                erows[b][t, pl.ds(k * 16, 16)] = \
                    (e[k] - mean) * y * g[k] + bt[k]

    stage_idx(0, 0)
    stage_idx(1, 1)
    stage_idx(2, 2)
    stage_pos(0, 0)
    stage_pos(1, 1)
    stage_tok(0)

    def body4(ch, carry):
        for j in range(NB):
            gi = NB * ch + j

            @pl.when(gi + 3 < ROWS_W)
            def _():
                stage_idx(gi + 3, (j + 3) % NB)

            @pl.when(gi + 2 < ROWS_W)
            def _():
                stage_pos(gi + 2, (j + 2) % NB)

            @pl.when(gi + 1 < ROWS_W)
            def _():
                stage_tok((j + 1) % NB)

            pltpu.make_async_copy(ttab_hbm.at[idxt[j]], erows[j],
                                  sem_t[j]).wait()
            compute_chunk(j)
            pltpu.async_copy(erows[j], out_hbm.at[base_b + gi], sem_o[j])
        return carry

    lax.fori_loop(0, ROWS_W // NB, body4, 0)

    for b in range(NB):
        pltpu.make_async_copy(erows[b], out_hbm.at[base_b], sem_o[b]).wait()


def kernel(input_token_id, input_position_id, token_table, pos_table,
           ln_gamma, ln_beta):
    tok = jnp.asarray(input_token_id, jnp.int32)
    pos = jnp.asarray(input_position_id, jnp.int32)
    ttab = jnp.pad(token_table, ((0, 0), (0, PAD - DIM)))
    ptab = jnp.pad(pos_table[:L], ((0, 0), (0, PAD - DIM)))
    out = _embed_ln_kernel(tok, pos, ttab, ptab, ln_gamma, ln_beta)
    return out[:, :, :DIM]
